# baseline (speedup 1.0000x reference)
import jax
import jax.numpy as jnp
from jax import lax
from jax.experimental import pallas as pl
from jax.experimental.pallas import tpu as pltpu
from jax.experimental.pallas import tpu_sc as plsc

_NC = 2
_NS = 16
_L = 16
_NW = _NC * _NS

_B, _S, _D = 4096, 200, 64
_F = _S * _D
_RPW = _B // _NW
_C = 4


def _body(x_hbm, pos_hbm, out_hbm, pos_v, buf, sem):
    wid = lax.axis_index("s") * _NC + lax.axis_index("c")
    base = wid * _RPW
    pltpu.sync_copy(pos_hbm, pos_v)

    def chunk(g, carry):
        row0 = base + g * _C
        pltpu.sync_copy(x_hbm.at[pl.ds(row0, _C)], buf)

        def add_i(i, c2):
            off = i * _L
            p = pos_v[pl.ds(off, _L)]
            for c in range(_C):
                buf[c, pl.ds(off, _L)] = buf[c, pl.ds(off, _L)] + p
            return c2

        lax.fori_loop(0, _F // _L, add_i, 0, unroll=4)
        pltpu.sync_copy(buf, out_hbm.at[pl.ds(row0, _C)])
        return carry

    lax.fori_loop(0, _RPW // _C, chunk, 0)


def kernel(x, pos_embedding):
    xf = x.reshape(_B, _F)
    posf = pos_embedding.reshape(_F)
    mesh = plsc.VectorSubcoreMesh(core_axis_name="c", subcore_axis_name="s")
    out = pl.kernel(
        _body,
        out_type=jax.ShapeDtypeStruct((_B, _F), jnp.float32),
        mesh=mesh,
        scratch_types=[
            pltpu.VMEM((_F,), jnp.float32),
            pltpu.VMEM((_C, _F), jnp.float32),
            pltpu.SemaphoreType.DMA,
        ],
    )(xf, posf)
    return out.reshape(_B, _S, _D)


# speedup vs baseline: 1.8681x; 1.8681x over previous
import jax
import jax.numpy as jnp
from jax import lax
from jax.experimental import pallas as pl
from jax.experimental.pallas import tpu as pltpu
from jax.experimental.pallas import tpu_sc as plsc

_NC = 2
_NS = 16
_L = 16
_NW = _NC * _NS

_B, _S, _D = 4096, 200, 64
_F = _S * _D
_RPW = _B // _NW
_C = 4


def _body(x_hbm, pos_hbm, out_hbm, pos_v, buf, sem):
    wid = lax.axis_index("s") * _NC + lax.axis_index("c")
    base = wid * _RPW
    pltpu.sync_copy(pos_hbm, pos_v)

    def chunk(g, carry):
        row0 = base + g * _C
        pltpu.sync_copy(x_hbm.at[pl.ds(row0, _C)], buf)

        def add_i(i, c2):
            off = i * _L
            p = pos_v[pl.ds(off, _L)]
            for c in range(_C):
                buf[c, pl.ds(off, _L)] = buf[c, pl.ds(off, _L)] + p
            return c2

        lax.fori_loop(0, _F // _L, add_i, 0, unroll=4)
        pltpu.sync_copy(buf, out_hbm.at[pl.ds(row0, _C)])
        return carry

    lax.fori_loop(0, _RPW // _C, chunk, 0)


def _tc_body(x_ref, pos_ref, o_ref):
    o_ref[...] = x_ref[...] + pos_ref[...]


_TCBLK = 128


def _tc_add(xf, posf2):
    return pl.pallas_call(
        _tc_body,
        grid=(_B // _TCBLK,),
        in_specs=[
            pl.BlockSpec((_TCBLK, _F), lambda i: (i, 0)),
            pl.BlockSpec((1, _F), lambda i: (0, 0)),
        ],
        out_specs=pl.BlockSpec((_TCBLK, _F), lambda i: (i, 0)),
        out_shape=jax.ShapeDtypeStruct((_B, _F), jnp.float32),
    )(xf, posf2)


def kernel(x, pos_embedding):
    return _tc_add(x.reshape(_B, _F), pos_embedding.reshape(1, _F)).reshape(_B, _S, _D)


def _sc_kernel_unused(x, pos_embedding):
    xf = x.reshape(_B, _F)
    posf = pos_embedding.reshape(_F)
    mesh = plsc.VectorSubcoreMesh(core_axis_name="c", subcore_axis_name="s")
    out = pl.kernel(
        _body,
        out_type=jax.ShapeDtypeStruct((_B, _F), jnp.float32),
        mesh=mesh,
        scratch_types=[
            pltpu.VMEM((_F,), jnp.float32),
            pltpu.VMEM((_C, _F), jnp.float32),
            pltpu.SemaphoreType.DMA,
        ],
    )(xf, posf)
    return out.reshape(_B, _S, _D)
